# Initial kernel scaffold; baseline (speedup 1.0000x reference)
import jax
import jax.numpy as jnp
from jax.experimental import pallas as pl


def kernel(x, temperature, embedding_weight):
    raise NotImplementedError("write your pallas kernel here")


# speedup vs baseline: 1.0560x; 1.0560x over previous
import functools

import jax
import jax.numpy as jnp
from jax import lax
from jax.experimental import pallas as pl
from jax.experimental.pallas import tpu as pltpu
from jax.experimental.pallas import tpu_sc as plsc

N_CODES = 8192
DIM = 64
M_TOKENS = 16384
BETA = 0.25

BM = 256
NBLK = M_TOKENS // BM

NC = 2
NS = 16
NW = NC * NS
B_PER_W = M_TOKENS // NW
IDX_CHUNK = 128
N_CHUNKS = B_PER_W // IDX_CHUNK


def _dist_argmin_block(x_ref, cb_ref, ids_ref, loss_ref, acc_ref):
    i = pl.program_id(0)
    x = x_ref[...]
    cb = cb_ref[...]
    xsq = jnp.sum(x * x, axis=1, keepdims=True)
    cbsq = jnp.sum(cb * cb, axis=1)
    mm = lax.dot_general(
        x, cb,
        dimension_numbers=(((1,), (1,)), ((), ())),
        preferred_element_type=jnp.float32,
    )
    dist = (xsq + cbsq[None, :]) - 2.0 * mm
    dmin = jnp.min(dist, axis=1, keepdims=True)
    col = lax.broadcasted_iota(jnp.int32, dist.shape, 1)
    ids = jnp.min(jnp.where(dist == dmin, col, N_CODES), axis=1)
    ids_ref[0, 0, :] = ids

    @pl.when(i == 0)
    def _():
        acc_ref[0, 0] = 0.0

    acc_ref[0, 0] += jnp.sum(dmin)

    @pl.when(i == pl.num_programs(0) - 1)
    def _():
        val = acc_ref[0, 0] * ((1.0 + BETA) / (M_TOKENS * DIM))
        loss_ref[...] = jnp.full((1, 1), val, jnp.float32)


def _sc_gather(table, idx):
    mesh = plsc.VectorSubcoreMesh(core_axis_name="c", subcore_axis_name="s")

    @functools.partial(
        pl.kernel,
        mesh=mesh,
        out_type=jax.ShapeDtypeStruct((M_TOKENS, DIM), jnp.float32),
        scratch_types=[
            pltpu.VMEM((B_PER_W,), jnp.int32),
            pltpu.VMEM((B_PER_W, DIM), jnp.float32),
            pltpu.SemaphoreType.DMA,
        ],
        compiler_params=pltpu.CompilerParams(use_tc_tiling_on_sc=False),
    )
    def gather_kernel(table_hbm, idx_hbm, out_hbm, idx_v, rows_v, sem):
        wid = lax.axis_index("s") * NC + lax.axis_index("c")
        base = wid * B_PER_W
        pltpu.sync_copy(idx_hbm.at[pl.ds(base, B_PER_W)], idx_v)
        copies = []
        for k in range(N_CHUNKS):
            copies.append(pltpu.async_copy(
                table_hbm.at[idx_v.at[pl.ds(k * IDX_CHUNK, IDX_CHUNK)]],
                rows_v.at[pl.ds(k * IDX_CHUNK, IDX_CHUNK)],
                sem,
            ))
        for c in copies:
            c.wait()
        pltpu.sync_copy(rows_v, out_hbm.at[pl.ds(base, B_PER_W)])

    return gather_kernel(table, idx)


def kernel(x, temperature, embedding_weight):
    del temperature
    ids3, loss2 = pl.pallas_call(
        _dist_argmin_block,
        grid=(NBLK,),
        in_specs=[
            pl.BlockSpec((BM, DIM), lambda i: (i, 0)),
            pl.BlockSpec((N_CODES, DIM), lambda i: (0, 0)),
        ],
        out_specs=[
            pl.BlockSpec((1, 1, BM), lambda i: (i, 0, 0)),
            pl.BlockSpec((1, 1), lambda i: (0, 0)),
        ],
        out_shape=[
            jax.ShapeDtypeStruct((NBLK, 1, BM), jnp.int32),
            jax.ShapeDtypeStruct((1, 1), jnp.float32),
        ],
        scratch_shapes=[pltpu.SMEM((1, 1), jnp.float32)],
    )(x, embedding_weight)
    ids = ids3.reshape(M_TOKENS)
    emb = _sc_gather(embedding_weight, ids)
    return emb, ids, loss2.reshape(())


# speedup vs baseline: 1.2591x; 1.1923x over previous
import functools

import jax
import jax.numpy as jnp
from jax import lax
from jax.experimental import pallas as pl
from jax.experimental.pallas import tpu as pltpu
from jax.experimental.pallas import tpu_sc as plsc

N_CODES = 8192
DIM = 64
M_TOKENS = 16384
BETA = 0.25

BM = 512
NBLK = M_TOKENS // BM

NC = 2
NS = 16
NW = NC * NS
B_PER_W = M_TOKENS // NW
IDX_CHUNK = 128
N_CHUNKS = B_PER_W // IDX_CHUNK


def _dist_argmin_block(x_ref, cb_ref, ids_ref, loss_ref, acc_ref):
    i = pl.program_id(0)
    x = x_ref[...]
    cb = cb_ref[...]
    xsq = jnp.sum(x * x, axis=1, keepdims=True)
    cbsq = jnp.sum(cb * cb, axis=1)
    mm = lax.dot_general(
        x, cb,
        dimension_numbers=(((1,), (1,)), ((), ())),
        preferred_element_type=jnp.float32,
    )
    dist = (xsq + cbsq[None, :]) - 2.0 * mm
    dmin = jnp.min(dist, axis=1, keepdims=True)
    col = lax.broadcasted_iota(jnp.int32, dist.shape, 1)
    ids = jnp.min(jnp.where(dist == dmin, col, N_CODES), axis=1)
    ids_ref[0, 0, :] = ids

    @pl.when(i == 0)
    def _():
        acc_ref[0, 0] = 0.0

    acc_ref[0, 0] += jnp.sum(dmin)

    @pl.when(i == pl.num_programs(0) - 1)
    def _():
        val = acc_ref[0, 0] * ((1.0 + BETA) / (M_TOKENS * DIM))
        loss_ref[...] = jnp.full((1, 1), val, jnp.float32)


def _sc_gather(table, idx):
    mesh = plsc.VectorSubcoreMesh(core_axis_name="c", subcore_axis_name="s")

    @functools.partial(
        pl.kernel,
        mesh=mesh,
        out_type=jax.ShapeDtypeStruct((M_TOKENS, DIM), jnp.float32),
        scratch_types=[
            pltpu.VMEM((B_PER_W,), jnp.int32),
            pltpu.VMEM((B_PER_W, DIM), jnp.float32),
            pltpu.SemaphoreType.DMA,
        ],
        compiler_params=pltpu.CompilerParams(use_tc_tiling_on_sc=False),
    )
    def gather_kernel(table_hbm, idx_hbm, out_hbm, idx_v, rows_v, sem):
        wid = lax.axis_index("s") * NC + lax.axis_index("c")
        base = wid * B_PER_W
        pltpu.sync_copy(idx_hbm.at[pl.ds(base, B_PER_W)], idx_v)
        copies = []
        for k in range(N_CHUNKS):
            copies.append(pltpu.async_copy(
                table_hbm.at[idx_v.at[pl.ds(k * IDX_CHUNK, IDX_CHUNK)]],
                rows_v.at[pl.ds(k * IDX_CHUNK, IDX_CHUNK)],
                sem,
            ))
        for c in copies:
            c.wait()
        pltpu.sync_copy(rows_v, out_hbm.at[pl.ds(base, B_PER_W)])

    return gather_kernel(table, idx)


def kernel(x, temperature, embedding_weight):
    del temperature
    ids3, loss2 = pl.pallas_call(
        _dist_argmin_block,
        grid=(NBLK,),
        in_specs=[
            pl.BlockSpec((BM, DIM), lambda i: (i, 0)),
            pl.BlockSpec((N_CODES, DIM), lambda i: (0, 0)),
        ],
        out_specs=[
            pl.BlockSpec((1, 1, BM), lambda i: (i, 0, 0)),
            pl.BlockSpec((1, 1), lambda i: (0, 0)),
        ],
        out_shape=[
            jax.ShapeDtypeStruct((NBLK, 1, BM), jnp.int32),
            jax.ShapeDtypeStruct((1, 1), jnp.float32),
        ],
        scratch_shapes=[pltpu.SMEM((1, 1), jnp.float32)],
    )(x, embedding_weight)
    ids = ids3.reshape(M_TOKENS)
    emb = _sc_gather(embedding_weight, ids)
    return emb, ids, loss2.reshape(())


# speedup vs baseline: 1.3236x; 1.0512x over previous
import functools

import jax
import jax.numpy as jnp
from jax import lax
from jax.experimental import pallas as pl
from jax.experimental.pallas import tpu as pltpu
from jax.experimental.pallas import tpu_sc as plsc

N_CODES = 8192
DIM = 64
M_TOKENS = 16384
BETA = 0.25

BM = 1024
NBLK = M_TOKENS // BM

NC = 2
NS = 16
NW = NC * NS
B_PER_W = M_TOKENS // NW
IDX_CHUNK = 128
N_CHUNKS = B_PER_W // IDX_CHUNK


def _dist_argmin_block(x_ref, cb_ref, ids_ref, loss_ref, acc_ref):
    i = pl.program_id(0)
    x = x_ref[...]
    cb = cb_ref[...]
    xsq = jnp.sum(x * x, axis=1, keepdims=True)
    cbsq = jnp.sum(cb * cb, axis=1)
    mm = lax.dot_general(
        x, cb,
        dimension_numbers=(((1,), (1,)), ((), ())),
        preferred_element_type=jnp.float32,
    )
    dist = (xsq + cbsq[None, :]) - 2.0 * mm
    dmin = jnp.min(dist, axis=1, keepdims=True)
    col = lax.broadcasted_iota(jnp.int32, dist.shape, 1)
    ids = jnp.min(jnp.where(dist == dmin, col, N_CODES), axis=1)
    ids_ref[0, 0, :] = ids

    @pl.when(i == 0)
    def _():
        acc_ref[0, 0] = 0.0

    acc_ref[0, 0] += jnp.sum(dmin)

    @pl.when(i == pl.num_programs(0) - 1)
    def _():
        val = acc_ref[0, 0] * ((1.0 + BETA) / (M_TOKENS * DIM))
        loss_ref[...] = jnp.full((1, 1), val, jnp.float32)


def _sc_gather(table, idx):
    mesh = plsc.VectorSubcoreMesh(core_axis_name="c", subcore_axis_name="s")

    @functools.partial(
        pl.kernel,
        mesh=mesh,
        out_type=jax.ShapeDtypeStruct((M_TOKENS, DIM), jnp.float32),
        scratch_types=[
            pltpu.VMEM((B_PER_W,), jnp.int32),
            pltpu.VMEM((B_PER_W, DIM), jnp.float32),
            pltpu.SemaphoreType.DMA,
        ],
        compiler_params=pltpu.CompilerParams(use_tc_tiling_on_sc=False),
    )
    def gather_kernel(table_hbm, idx_hbm, out_hbm, idx_v, rows_v, sem):
        wid = lax.axis_index("s") * NC + lax.axis_index("c")
        base = wid * B_PER_W
        pltpu.sync_copy(idx_hbm.at[pl.ds(base, B_PER_W)], idx_v)
        copies = []
        for k in range(N_CHUNKS):
            copies.append(pltpu.async_copy(
                table_hbm.at[idx_v.at[pl.ds(k * IDX_CHUNK, IDX_CHUNK)]],
                rows_v.at[pl.ds(k * IDX_CHUNK, IDX_CHUNK)],
                sem,
            ))
        for c in copies:
            c.wait()
        pltpu.sync_copy(rows_v, out_hbm.at[pl.ds(base, B_PER_W)])

    return gather_kernel(table, idx)


def kernel(x, temperature, embedding_weight):
    del temperature
    ids3, loss2 = pl.pallas_call(
        _dist_argmin_block,
        grid=(NBLK,),
        in_specs=[
            pl.BlockSpec((BM, DIM), lambda i: (i, 0)),
            pl.BlockSpec((N_CODES, DIM), lambda i: (0, 0)),
        ],
        out_specs=[
            pl.BlockSpec((1, 1, BM), lambda i: (i, 0, 0)),
            pl.BlockSpec((1, 1), lambda i: (0, 0)),
        ],
        out_shape=[
            jax.ShapeDtypeStruct((NBLK, 1, BM), jnp.int32),
            jax.ShapeDtypeStruct((1, 1), jnp.float32),
        ],
        scratch_shapes=[pltpu.SMEM((1, 1), jnp.float32)],
    )(x, embedding_weight)
    ids = ids3.reshape(M_TOKENS)
    emb = _sc_gather(embedding_weight, ids)
    return emb, ids, loss2.reshape(())
